# baseline (speedup 1.0000x reference)
import dataclasses
import functools

import jax
import jax.numpy as jnp
from jax import lax
from jax.experimental import pallas as pl
from jax.experimental.pallas import tpu as pltpu
from jax.experimental.pallas import tpu_sc as plsc

N_NODES = 10000
D = 128
E_EDGES = 320000

NC, NS = 2, 16
NW = NC * NS
K = 56
CH0 = 240
CH1 = 120
CH = (CH0 + CH1) // 2
GB = 4
SB = 2
ED = 8
U = 8
E_PAD = NW * CH * K
WPT = 632
WPT_LAST = N_NODES - 15 * WPT
N_DEG = 10240
DEG_PER_TILE = N_DEG // NS

_sc_mesh = plsc.VectorSubcoreMesh(core_axis_name="c", subcore_axis_name="s")

_sc_params = pltpu.CompilerParams()
if "needs_layout_passes" in pltpu.CompilerParams.__dataclass_fields__:
    _sc_params = dataclasses.replace(_sc_params, needs_layout_passes=False)


def _deg_body(dst_hbm, w_hbm, zd_hbm, out_hbm, dst_v, w_v, deg_sh, sem):
    c = lax.axis_index("c")
    s = lax.axis_index("s")
    z0 = s * DEG_PER_TILE
    pltpu.async_copy(zd_hbm.at[pl.ds(z0, DEG_PER_TILE)],
                     deg_sh.at[pl.ds(z0, DEG_PER_TILE)], sem).wait()
    chc = jnp.where(c == 0, CH0, CH1)
    row0 = c * NS * CH0 + s * chc

    @pl.when(c == 0)
    def _():
        pltpu.sync_copy(dst_hbm.at[pl.ds(row0, CH0)], dst_v.at[pl.ds(0, CH0)])
        pltpu.sync_copy(w_hbm.at[pl.ds(row0, CH0)], w_v.at[pl.ds(0, CH0)])

    @pl.when(c == 1)
    def _():
        pltpu.sync_copy(dst_hbm.at[pl.ds(row0, CH1)], dst_v.at[pl.ds(0, CH1)])
        pltpu.sync_copy(w_hbm.at[pl.ds(row0, CH1)], w_v.at[pl.ds(0, CH1)])

    plsc.subcore_barrier()

    @pl.loop(0, chc)
    def _(j):
        pltpu.sync_copy(w_v.at[j], deg_sh.at[dst_v.at[j]], add=True)

    plsc.subcore_barrier()
    pltpu.async_copy(deg_sh.at[pl.ds(z0, DEG_PER_TILE)],
                     out_hbm.at[c].at[pl.ds(z0, DEG_PER_TILE)], sem).wait()


@jax.jit
def _sc_degree(dst2d, w2d, zd):
    return pl.kernel(
        _deg_body,
        out_type=jax.ShapeDtypeStruct((NC, N_DEG), jnp.float32),
        mesh=_sc_mesh,
        scratch_types=[
            pltpu.VMEM((CH0, K), jnp.int32),
            pltpu.VMEM((CH0, K), jnp.float32),
            pltpu.VMEM_SHARED((N_DEG,), jnp.float32),
            pltpu.SemaphoreType.DMA,
        ],
    )(dst2d, w2d, zd)


def _load_erow(src_hbm, dst_hbm, w_hbm, src_v, dst_v, w_v, row0, j, q, sem):
    pltpu.async_copy(src_hbm.at[pl.ds(row0 + j, 1)],
                     src_v.at[pl.ds(q, 1)], sem)
    pltpu.async_copy(dst_hbm.at[pl.ds(row0 + j, 1)],
                     dst_v.at[pl.ds(q, 1)], sem)
    pltpu.async_copy(w_hbm.at[pl.ds(row0 + j, 1)],
                     w_v.at[pl.ds(q, 1)], sem)


def _wait_erow(src_hbm, dst_hbm, w_hbm, src_v, dst_v, w_v, row0, q, sem):
    pltpu.make_async_copy(src_hbm.at[pl.ds(row0, 1)],
                          src_v.at[pl.ds(q, 1)], sem).wait()
    pltpu.make_async_copy(dst_hbm.at[pl.ds(row0, 1)],
                          dst_v.at[pl.ds(q, 1)], sem).wait()
    pltpu.make_async_copy(w_hbm.at[pl.ds(row0, 1)],
                          w_v.at[pl.ds(q, 1)], sem).wait()


def _agg_body(g_hbm, src_hbm, dst_hbm, w_hbm, zeros_hbm, out_hbm,
              src_v, dst_v, w_v, gbuf, sbuf, acc_sh, *sems):
    gsems = sems[0:GB]
    ssems = sems[GB:GB + SB]
    esems = sems[GB + SB:GB + SB + ED]
    msem = sems[GB + SB + ED]
    c = lax.axis_index("c")
    s = lax.axis_index("s")
    z0 = s * WPT

    @pl.when(s < NS - 1)
    def _():
        pltpu.async_copy(zeros_hbm.at[pl.ds(z0, WPT)],
                         acc_sh.at[pl.ds(z0, WPT)], msem)

    @pl.when(s == NS - 1)
    def _():
        pltpu.async_copy(zeros_hbm.at[pl.ds(z0, WPT_LAST)],
                         acc_sh.at[pl.ds(z0, WPT_LAST)], msem)

    chc = jnp.where(c == 0, CH0, CH1)
    row0 = c * NS * CH0 + s * chc
    pltpu.sync_copy(src_hbm.at[pl.ds(row0, GB)], src_v.at[pl.ds(0, GB)])
    pltpu.sync_copy(dst_hbm.at[pl.ds(row0, GB)], dst_v.at[pl.ds(0, GB)])
    pltpu.sync_copy(w_hbm.at[pl.ds(row0, GB)], w_v.at[pl.ds(0, GB)])
    for t in (GB, GB + 1):
        _load_erow(src_hbm, dst_hbm, w_hbm, src_v, dst_v, w_v,
                   row0, t, t % ED, esems[t % ED])

    @pl.when(s < NS - 1)
    def _():
        pltpu.make_async_copy(zeros_hbm.at[pl.ds(z0, WPT)],
                              acc_sh.at[pl.ds(z0, WPT)], msem).wait()

    @pl.when(s == NS - 1)
    def _():
        pltpu.make_async_copy(zeros_hbm.at[pl.ds(z0, WPT_LAST)],
                              acc_sh.at[pl.ds(z0, WPT_LAST)], msem).wait()

    plsc.subcore_barrier()

    for b in range(GB):
        pltpu.async_copy(g_hbm.at[src_v.at[b]], gbuf.at[b], gsems[b])

    @pl.loop(0, chc // U)
    def _(jj):
        for t in range(U):
            j = jj * U + t
            gb = t % GB
            sb = t % SB
            ql = (t + GB + 2) % ED
            qg = (t + GB) % ED
            pltpu.make_async_copy(g_hbm.at[src_v.at[t % ED]], gbuf.at[gb],
                                  gsems[gb]).wait()

            @pl.when(j >= SB)
            def _():
                pltpu.make_async_copy(sbuf.at[sb],
                                      acc_sh.at[dst_v.at[t % ED]],
                                      ssems[sb]).wait()

            @pl.when(j + GB + 2 < chc)
            def _():
                _load_erow(src_hbm, dst_hbm, w_hbm, src_v, dst_v, w_v,
                           row0, j + GB + 2, ql, esems[ql])

            @pl.loop(0, K)
            def _(i):
                wspl = plsc.load_gather(w_v.at[t % ED],
                                        [jnp.full((16,), i, jnp.int32)])
                for cb in range(D // 16):
                    sl = (i, pl.ds(cb * 16, 16))
                    sbuf.at[sb][sl] = gbuf.at[gb][sl] * wspl

            @pl.when(j + GB < chc)
            def _():
                _wait_erow(src_hbm, dst_hbm, w_hbm, src_v, dst_v, w_v,
                           row0, qg, esems[qg])
                pltpu.async_copy(g_hbm.at[src_v.at[qg]], gbuf.at[gb],
                                 gsems[gb])

            pltpu.async_copy(sbuf.at[sb], acc_sh.at[dst_v.at[t % ED]],
                             ssems[sb], add=True)

    for r in range(SB):
        pltpu.make_async_copy(sbuf.at[r], acc_sh.at[dst_v.at[r]],
                              ssems[r]).wait()
    plsc.subcore_barrier()

    @pl.when(s < NS - 1)
    def _():
        pltpu.async_copy(acc_sh.at[pl.ds(z0, WPT)],
                         out_hbm.at[c].at[pl.ds(z0, WPT)], msem).wait()

    @pl.when(s == NS - 1)
    def _():
        pltpu.async_copy(acc_sh.at[pl.ds(z0, WPT_LAST)],
                         out_hbm.at[c].at[pl.ds(z0, WPT_LAST)], msem).wait()


@jax.jit
def _sc_aggregate(g, src2d, dst2d, w2d, zeros):
    return pl.kernel(
        _agg_body,
        out_type=jax.ShapeDtypeStruct((NC, N_NODES, D), jnp.float32),
        mesh=_sc_mesh,
        scratch_types=[
            pltpu.VMEM((ED, K), jnp.int32),
            pltpu.VMEM((ED, K), jnp.int32),
            pltpu.VMEM((ED, K), jnp.float32),
            pltpu.VMEM((GB, K, D), jnp.float32),
            pltpu.VMEM((SB, K, D), jnp.float32),
            pltpu.VMEM_SHARED((N_NODES, D), jnp.float32),
        ] + [pltpu.SemaphoreType.DMA] * (GB + SB + ED + 1),
        compiler_params=_sc_params,
    )(g, src2d, dst2d, w2d, zeros)


_BLK = 2000


def _mm_body(x_ref, w_ref, o_ref):
    o_ref[...] = jnp.dot(x_ref[...], w_ref[...],
                         preferred_element_type=jnp.float32)


def _matmul(x, w):
    return pl.pallas_call(
        _mm_body,
        grid=(N_NODES // _BLK,),
        in_specs=[pl.BlockSpec((_BLK, D), lambda i: (i, 0)),
                  pl.BlockSpec((D, D), lambda i: (0, 0))],
        out_specs=pl.BlockSpec((_BLK, D), lambda i: (i, 0)),
        out_shape=jax.ShapeDtypeStruct((N_NODES, D), jnp.float32),
    )(x, w)


def _prep_body(deg_ref, h_ref, dis_ref, g_ref):
    d = deg_ref[:, 0:1] + deg_ref[:, 1:2] + 1.0
    di = lax.rsqrt(d)
    dis_ref[...] = di
    g_ref[...] = di * h_ref[...]


def _tc_prep(deg01, h):
    return pl.pallas_call(
        _prep_body,
        grid=(N_NODES // _BLK,),
        in_specs=[pl.BlockSpec((_BLK, NC), lambda i: (i, 0)),
                  pl.BlockSpec((_BLK, D), lambda i: (i, 0))],
        out_specs=[pl.BlockSpec((_BLK, 1), lambda i: (i, 0)),
                   pl.BlockSpec((_BLK, D), lambda i: (i, 0))],
        out_shape=[jax.ShapeDtypeStruct((N_NODES, 1), jnp.float32),
                   jax.ShapeDtypeStruct((N_NODES, D), jnp.float32)],
    )(deg01, h)


def _mid_body(acc_ref, g_ref, dis_ref, b_ref, w_ref, g2_ref):
    di = dis_ref[...]
    a = acc_ref[0] + acc_ref[1] + g_ref[...]
    h1 = jnp.maximum(di * a + b_ref[...], 0.0)
    g2_ref[...] = di * jnp.dot(h1, w_ref[...],
                               preferred_element_type=jnp.float32)


def _tc_mid(acc_p, g, dis, b1, W2):
    return pl.pallas_call(
        _mid_body,
        grid=(N_NODES // _BLK,),
        in_specs=[pl.BlockSpec((NC, _BLK, D), lambda i: (0, i, 0)),
                  pl.BlockSpec((_BLK, D), lambda i: (i, 0)),
                  pl.BlockSpec((_BLK, 1), lambda i: (i, 0)),
                  pl.BlockSpec((1, D), lambda i: (0, 0)),
                  pl.BlockSpec((D, D), lambda i: (0, 0))],
        out_specs=pl.BlockSpec((_BLK, D), lambda i: (i, 0)),
        out_shape=jax.ShapeDtypeStruct((N_NODES, D), jnp.float32),
    )(acc_p, g, dis, b1.reshape(1, D), W2)


def _final_body(acc_ref, g_ref, dis_ref, b_ref, o_ref):
    di = dis_ref[...]
    a = acc_ref[0] + acc_ref[1] + g_ref[...]
    o_ref[...] = di * a + b_ref[...]


def _tc_final(acc_p, g2, dis, b2):
    return pl.pallas_call(
        _final_body,
        grid=(N_NODES // _BLK,),
        in_specs=[pl.BlockSpec((NC, _BLK, D), lambda i: (0, i, 0)),
                  pl.BlockSpec((_BLK, D), lambda i: (i, 0)),
                  pl.BlockSpec((_BLK, 1), lambda i: (i, 0)),
                  pl.BlockSpec((1, D), lambda i: (0, 0))],
        out_specs=pl.BlockSpec((_BLK, D), lambda i: (i, 0)),
        out_shape=jax.ShapeDtypeStruct((N_NODES, D), jnp.float32),
    )(acc_p, g2, dis, b2.reshape(1, D))


def kernel(x, edge_index, edge_attr, W1, b1, W2, b2):
    src = edge_index[0]
    dst = edge_index[1]
    pad = E_PAD - E_EDGES
    src2d = jnp.concatenate(
        [src, jnp.zeros((pad,), jnp.int32)]).reshape(NW * CH, K)
    dst2d = jnp.concatenate(
        [dst, jnp.zeros((pad,), jnp.int32)]).reshape(NW * CH, K)
    w2d = jnp.concatenate(
        [edge_attr, jnp.zeros((pad,), jnp.float32)]).reshape(NW * CH, K)
    zeros = jnp.zeros((N_NODES, D), jnp.float32)
    zd = jnp.zeros((N_DEG,), jnp.float32)

    deg01 = _sc_degree(dst2d, w2d, zd)[:, :N_NODES].T
    h = _matmul(x, W1)
    dis, g1 = _tc_prep(deg01, h)

    acc1 = _sc_aggregate(g1, src2d, dst2d, w2d, zeros)
    g2 = _tc_mid(acc1, g1, dis, b1, W2)
    acc2 = _sc_aggregate(g2, src2d, dst2d, w2d, zeros)
    return _tc_final(acc2, g2, dis, b2)


# speedup vs baseline: 1.0628x; 1.0628x over previous
import dataclasses
import functools

import jax
import jax.numpy as jnp
from jax import lax
from jax.experimental import pallas as pl
from jax.experimental.pallas import tpu as pltpu
from jax.experimental.pallas import tpu_sc as plsc

N_NODES = 10000
D = 128
E_EDGES = 320000

NC, NS = 2, 16
NW = NC * NS
K = 56
CH0 = 224
CH1 = 136
CH = (CH0 + CH1) // 2
GB = 4
SB = 2
ED = 8
U = 8
E_PAD = NW * CH * K
WPT = 632
WPT_LAST = N_NODES - 15 * WPT
N_DEG = 10240
DEG_PER_TILE = N_DEG // NS

_sc_mesh = plsc.VectorSubcoreMesh(core_axis_name="c", subcore_axis_name="s")

_sc_params = pltpu.CompilerParams()
if "needs_layout_passes" in pltpu.CompilerParams.__dataclass_fields__:
    _sc_params = dataclasses.replace(_sc_params, needs_layout_passes=False)


def _deg_body(dst_hbm, w_hbm, zd_hbm, out_hbm, dst_v, w_v, deg_sh, sem):
    c = lax.axis_index("c")
    s = lax.axis_index("s")
    z0 = s * DEG_PER_TILE
    pltpu.async_copy(zd_hbm.at[pl.ds(z0, DEG_PER_TILE)],
                     deg_sh.at[pl.ds(z0, DEG_PER_TILE)], sem).wait()
    chc = jnp.where(c == 0, CH0, CH1)
    row0 = c * NS * CH0 + s * chc

    @pl.when(c == 0)
    def _():
        pltpu.sync_copy(dst_hbm.at[pl.ds(row0, CH0)], dst_v.at[pl.ds(0, CH0)])
        pltpu.sync_copy(w_hbm.at[pl.ds(row0, CH0)], w_v.at[pl.ds(0, CH0)])

    @pl.when(c == 1)
    def _():
        pltpu.sync_copy(dst_hbm.at[pl.ds(row0, CH1)], dst_v.at[pl.ds(0, CH1)])
        pltpu.sync_copy(w_hbm.at[pl.ds(row0, CH1)], w_v.at[pl.ds(0, CH1)])

    plsc.subcore_barrier()

    @pl.loop(0, chc)
    def _(j):
        pltpu.sync_copy(w_v.at[j], deg_sh.at[dst_v.at[j]], add=True)

    plsc.subcore_barrier()
    pltpu.async_copy(deg_sh.at[pl.ds(z0, DEG_PER_TILE)],
                     out_hbm.at[c].at[pl.ds(z0, DEG_PER_TILE)], sem).wait()


@jax.jit
def _sc_degree(dst2d, w2d, zd):
    return pl.kernel(
        _deg_body,
        out_type=jax.ShapeDtypeStruct((NC, N_DEG), jnp.float32),
        mesh=_sc_mesh,
        scratch_types=[
            pltpu.VMEM((CH0, K), jnp.int32),
            pltpu.VMEM((CH0, K), jnp.float32),
            pltpu.VMEM_SHARED((N_DEG,), jnp.float32),
            pltpu.SemaphoreType.DMA,
        ],
    )(dst2d, w2d, zd)


def _load_erow(src_hbm, dst_hbm, w_hbm, src_v, dst_v, w_v, row0, j, q, sem):
    pltpu.async_copy(src_hbm.at[pl.ds(row0 + j, 1)],
                     src_v.at[pl.ds(q, 1)], sem)
    pltpu.async_copy(dst_hbm.at[pl.ds(row0 + j, 1)],
                     dst_v.at[pl.ds(q, 1)], sem)
    pltpu.async_copy(w_hbm.at[pl.ds(row0 + j, 1)],
                     w_v.at[pl.ds(q, 1)], sem)


def _wait_erow(src_hbm, dst_hbm, w_hbm, src_v, dst_v, w_v, row0, q, sem):
    pltpu.make_async_copy(src_hbm.at[pl.ds(row0, 1)],
                          src_v.at[pl.ds(q, 1)], sem).wait()
    pltpu.make_async_copy(dst_hbm.at[pl.ds(row0, 1)],
                          dst_v.at[pl.ds(q, 1)], sem).wait()
    pltpu.make_async_copy(w_hbm.at[pl.ds(row0, 1)],
                          w_v.at[pl.ds(q, 1)], sem).wait()


def _agg_body(g_hbm, src_hbm, dst_hbm, w_hbm, zeros_hbm, out_hbm,
              src_v, dst_v, w_v, gbuf, sbuf, acc_sh, *sems):
    gsems = sems[0:GB]
    ssems = sems[GB:GB + SB]
    esems = sems[GB + SB:GB + SB + ED]
    msem = sems[GB + SB + ED]
    c = lax.axis_index("c")
    s = lax.axis_index("s")
    z0 = s * WPT

    @pl.when(s < NS - 1)
    def _():
        pltpu.async_copy(zeros_hbm.at[pl.ds(z0, WPT)],
                         acc_sh.at[pl.ds(z0, WPT)], msem)

    @pl.when(s == NS - 1)
    def _():
        pltpu.async_copy(zeros_hbm.at[pl.ds(z0, WPT_LAST)],
                         acc_sh.at[pl.ds(z0, WPT_LAST)], msem)

    chc = jnp.where(c == 0, CH0, CH1)
    row0 = c * NS * CH0 + s * chc
    pltpu.sync_copy(src_hbm.at[pl.ds(row0, GB)], src_v.at[pl.ds(0, GB)])
    pltpu.sync_copy(dst_hbm.at[pl.ds(row0, GB)], dst_v.at[pl.ds(0, GB)])
    pltpu.sync_copy(w_hbm.at[pl.ds(row0, GB)], w_v.at[pl.ds(0, GB)])
    for t in (GB, GB + 1):
        _load_erow(src_hbm, dst_hbm, w_hbm, src_v, dst_v, w_v,
                   row0, t, t % ED, esems[t % ED])

    @pl.when(s < NS - 1)
    def _():
        pltpu.make_async_copy(zeros_hbm.at[pl.ds(z0, WPT)],
                              acc_sh.at[pl.ds(z0, WPT)], msem).wait()

    @pl.when(s == NS - 1)
    def _():
        pltpu.make_async_copy(zeros_hbm.at[pl.ds(z0, WPT_LAST)],
                              acc_sh.at[pl.ds(z0, WPT_LAST)], msem).wait()

    plsc.subcore_barrier()

    for b in range(GB):
        pltpu.async_copy(g_hbm.at[src_v.at[b]], gbuf.at[b], gsems[b])

    @pl.loop(0, chc // U)
    def _(jj):
        for t in range(U):
            j = jj * U + t
            gb = t % GB
            sb = t % SB
            ql = (t + GB + 2) % ED
            qg = (t + GB) % ED
            pltpu.make_async_copy(g_hbm.at[src_v.at[t % ED]], gbuf.at[gb],
                                  gsems[gb]).wait()

            @pl.when(j >= SB)
            def _():
                pltpu.make_async_copy(sbuf.at[sb],
                                      acc_sh.at[dst_v.at[t % ED]],
                                      ssems[sb]).wait()

            @pl.when(j + GB + 2 < chc)
            def _():
                _load_erow(src_hbm, dst_hbm, w_hbm, src_v, dst_v, w_v,
                           row0, j + GB + 2, ql, esems[ql])

            @pl.loop(0, K)
            def _(i):
                wspl = plsc.load_gather(w_v.at[t % ED],
                                        [jnp.full((16,), i, jnp.int32)])
                for cb in range(D // 16):
                    sl = (i, pl.ds(cb * 16, 16))
                    sbuf.at[sb][sl] = gbuf.at[gb][sl] * wspl

            @pl.when(j + GB < chc)
            def _():
                _wait_erow(src_hbm, dst_hbm, w_hbm, src_v, dst_v, w_v,
                           row0, qg, esems[qg])
                pltpu.async_copy(g_hbm.at[src_v.at[qg]], gbuf.at[gb],
                                 gsems[gb])

            pltpu.async_copy(sbuf.at[sb], acc_sh.at[dst_v.at[t % ED]],
                             ssems[sb], add=True)

    for r in range(SB):
        pltpu.make_async_copy(sbuf.at[r], acc_sh.at[dst_v.at[r]],
                              ssems[r]).wait()
    plsc.subcore_barrier()

    @pl.when(s < NS - 1)
    def _():
        pltpu.async_copy(acc_sh.at[pl.ds(z0, WPT)],
                         out_hbm.at[c].at[pl.ds(z0, WPT)], msem).wait()

    @pl.when(s == NS - 1)
    def _():
        pltpu.async_copy(acc_sh.at[pl.ds(z0, WPT_LAST)],
                         out_hbm.at[c].at[pl.ds(z0, WPT_LAST)], msem).wait()


@jax.jit
def _sc_aggregate(g, src2d, dst2d, w2d, zeros):
    return pl.kernel(
        _agg_body,
        out_type=jax.ShapeDtypeStruct((NC, N_NODES, D), jnp.float32),
        mesh=_sc_mesh,
        scratch_types=[
            pltpu.VMEM((ED, K), jnp.int32),
            pltpu.VMEM((ED, K), jnp.int32),
            pltpu.VMEM((ED, K), jnp.float32),
            pltpu.VMEM((GB, K, D), jnp.float32),
            pltpu.VMEM((SB, K, D), jnp.float32),
            pltpu.VMEM_SHARED((N_NODES, D), jnp.float32),
        ] + [pltpu.SemaphoreType.DMA] * (GB + SB + ED + 1),
        compiler_params=_sc_params,
    )(g, src2d, dst2d, w2d, zeros)


_BLK = 2000


def _mm_body(x_ref, w_ref, o_ref):
    o_ref[...] = jnp.dot(x_ref[...], w_ref[...],
                         preferred_element_type=jnp.float32)


def _matmul(x, w):
    return pl.pallas_call(
        _mm_body,
        grid=(N_NODES // _BLK,),
        in_specs=[pl.BlockSpec((_BLK, D), lambda i: (i, 0)),
                  pl.BlockSpec((D, D), lambda i: (0, 0))],
        out_specs=pl.BlockSpec((_BLK, D), lambda i: (i, 0)),
        out_shape=jax.ShapeDtypeStruct((N_NODES, D), jnp.float32),
    )(x, w)


def _prep_body(deg_ref, h_ref, dis_ref, g_ref):
    d = deg_ref[:, 0:1] + deg_ref[:, 1:2] + 1.0
    di = lax.rsqrt(d)
    dis_ref[...] = di
    g_ref[...] = di * h_ref[...]


def _tc_prep(deg01, h):
    return pl.pallas_call(
        _prep_body,
        grid=(N_NODES // _BLK,),
        in_specs=[pl.BlockSpec((_BLK, NC), lambda i: (i, 0)),
                  pl.BlockSpec((_BLK, D), lambda i: (i, 0))],
        out_specs=[pl.BlockSpec((_BLK, 1), lambda i: (i, 0)),
                   pl.BlockSpec((_BLK, D), lambda i: (i, 0))],
        out_shape=[jax.ShapeDtypeStruct((N_NODES, 1), jnp.float32),
                   jax.ShapeDtypeStruct((N_NODES, D), jnp.float32)],
    )(deg01, h)


def _mid_body(acc_ref, g_ref, dis_ref, b_ref, w_ref, g2_ref):
    di = dis_ref[...]
    a = acc_ref[0] + acc_ref[1] + g_ref[...]
    h1 = jnp.maximum(di * a + b_ref[...], 0.0)
    g2_ref[...] = di * jnp.dot(h1, w_ref[...],
                               preferred_element_type=jnp.float32)


def _tc_mid(acc_p, g, dis, b1, W2):
    return pl.pallas_call(
        _mid_body,
        grid=(N_NODES // _BLK,),
        in_specs=[pl.BlockSpec((NC, _BLK, D), lambda i: (0, i, 0)),
                  pl.BlockSpec((_BLK, D), lambda i: (i, 0)),
                  pl.BlockSpec((_BLK, 1), lambda i: (i, 0)),
                  pl.BlockSpec((1, D), lambda i: (0, 0)),
                  pl.BlockSpec((D, D), lambda i: (0, 0))],
        out_specs=pl.BlockSpec((_BLK, D), lambda i: (i, 0)),
        out_shape=jax.ShapeDtypeStruct((N_NODES, D), jnp.float32),
    )(acc_p, g, dis, b1.reshape(1, D), W2)


def _final_body(acc_ref, g_ref, dis_ref, b_ref, o_ref):
    di = dis_ref[...]
    a = acc_ref[0] + acc_ref[1] + g_ref[...]
    o_ref[...] = di * a + b_ref[...]


def _tc_final(acc_p, g2, dis, b2):
    return pl.pallas_call(
        _final_body,
        grid=(N_NODES // _BLK,),
        in_specs=[pl.BlockSpec((NC, _BLK, D), lambda i: (0, i, 0)),
                  pl.BlockSpec((_BLK, D), lambda i: (i, 0)),
                  pl.BlockSpec((_BLK, 1), lambda i: (i, 0)),
                  pl.BlockSpec((1, D), lambda i: (0, 0))],
        out_specs=pl.BlockSpec((_BLK, D), lambda i: (i, 0)),
        out_shape=jax.ShapeDtypeStruct((N_NODES, D), jnp.float32),
    )(acc_p, g2, dis, b2.reshape(1, D))


def kernel(x, edge_index, edge_attr, W1, b1, W2, b2):
    src = edge_index[0]
    dst = edge_index[1]
    pad = E_PAD - E_EDGES
    src2d = jnp.concatenate(
        [src, jnp.zeros((pad,), jnp.int32)]).reshape(NW * CH, K)
    dst2d = jnp.concatenate(
        [dst, jnp.zeros((pad,), jnp.int32)]).reshape(NW * CH, K)
    w2d = jnp.concatenate(
        [edge_attr, jnp.zeros((pad,), jnp.float32)]).reshape(NW * CH, K)
    zeros = jnp.zeros((N_NODES, D), jnp.float32)
    zd = jnp.zeros((N_DEG,), jnp.float32)

    deg01 = _sc_degree(dst2d, w2d, zd)[:, :N_NODES].T
    h = _matmul(x, W1)
    dis, g1 = _tc_prep(deg01, h)

    acc1 = _sc_aggregate(g1, src2d, dst2d, w2d, zeros)
    g2 = _tc_mid(acc1, g1, dis, b1, W2)
    acc2 = _sc_aggregate(g2, src2d, dst2d, w2d, zeros)
    return _tc_final(acc2, g2, dis, b2)


# speedup vs baseline: 1.0975x; 1.0326x over previous
import dataclasses
import functools

import jax
import jax.numpy as jnp
from jax import lax
from jax.experimental import pallas as pl
from jax.experimental.pallas import tpu as pltpu
from jax.experimental.pallas import tpu_sc as plsc

N_NODES = 10000
D = 128
E_EDGES = 320000

NC, NS = 2, 16
NW = NC * NS
K = 56
CH0 = 216
CH1 = 144
CH = (CH0 + CH1) // 2
GB = 4
SB = 2
ED = 8
U = 8
E_PAD = NW * CH * K
WPT = 632
WPT_LAST = N_NODES - 15 * WPT
N_DEG = 10240
DEG_PER_TILE = N_DEG // NS

_sc_mesh = plsc.VectorSubcoreMesh(core_axis_name="c", subcore_axis_name="s")

_sc_params = pltpu.CompilerParams()
if "needs_layout_passes" in pltpu.CompilerParams.__dataclass_fields__:
    _sc_params = dataclasses.replace(_sc_params, needs_layout_passes=False)


def _deg_body(dst_hbm, w_hbm, zd_hbm, out_hbm, dst_v, w_v, deg_sh, sem):
    c = lax.axis_index("c")
    s = lax.axis_index("s")
    z0 = s * DEG_PER_TILE
    pltpu.async_copy(zd_hbm.at[pl.ds(z0, DEG_PER_TILE)],
                     deg_sh.at[pl.ds(z0, DEG_PER_TILE)], sem).wait()
    chc = jnp.where(c == 0, CH0, CH1)
    row0 = c * NS * CH0 + s * chc

    @pl.when(c == 0)
    def _():
        pltpu.sync_copy(dst_hbm.at[pl.ds(row0, CH0)], dst_v.at[pl.ds(0, CH0)])
        pltpu.sync_copy(w_hbm.at[pl.ds(row0, CH0)], w_v.at[pl.ds(0, CH0)])

    @pl.when(c == 1)
    def _():
        pltpu.sync_copy(dst_hbm.at[pl.ds(row0, CH1)], dst_v.at[pl.ds(0, CH1)])
        pltpu.sync_copy(w_hbm.at[pl.ds(row0, CH1)], w_v.at[pl.ds(0, CH1)])

    plsc.subcore_barrier()

    @pl.loop(0, chc)
    def _(j):
        pltpu.sync_copy(w_v.at[j], deg_sh.at[dst_v.at[j]], add=True)

    plsc.subcore_barrier()
    pltpu.async_copy(deg_sh.at[pl.ds(z0, DEG_PER_TILE)],
                     out_hbm.at[c].at[pl.ds(z0, DEG_PER_TILE)], sem).wait()


@jax.jit
def _sc_degree(dst2d, w2d, zd):
    return pl.kernel(
        _deg_body,
        out_type=jax.ShapeDtypeStruct((NC, N_DEG), jnp.float32),
        mesh=_sc_mesh,
        scratch_types=[
            pltpu.VMEM((CH0, K), jnp.int32),
            pltpu.VMEM((CH0, K), jnp.float32),
            pltpu.VMEM_SHARED((N_DEG,), jnp.float32),
            pltpu.SemaphoreType.DMA,
        ],
    )(dst2d, w2d, zd)


def _load_erow(src_hbm, dst_hbm, w_hbm, src_v, dst_v, w_v, row0, j, q, sem):
    pltpu.async_copy(src_hbm.at[pl.ds(row0 + j, 1)],
                     src_v.at[pl.ds(q, 1)], sem)
    pltpu.async_copy(dst_hbm.at[pl.ds(row0 + j, 1)],
                     dst_v.at[pl.ds(q, 1)], sem)
    pltpu.async_copy(w_hbm.at[pl.ds(row0 + j, 1)],
                     w_v.at[pl.ds(q, 1)], sem)


def _wait_erow(src_hbm, dst_hbm, w_hbm, src_v, dst_v, w_v, row0, q, sem):
    pltpu.make_async_copy(src_hbm.at[pl.ds(row0, 1)],
                          src_v.at[pl.ds(q, 1)], sem).wait()
    pltpu.make_async_copy(dst_hbm.at[pl.ds(row0, 1)],
                          dst_v.at[pl.ds(q, 1)], sem).wait()
    pltpu.make_async_copy(w_hbm.at[pl.ds(row0, 1)],
                          w_v.at[pl.ds(q, 1)], sem).wait()


def _agg_body(g_hbm, src_hbm, dst_hbm, w_hbm, zeros_hbm, out_hbm,
              src_v, dst_v, w_v, gbuf, sbuf, acc_sh, *sems):
    gsems = sems[0:GB]
    ssems = sems[GB:GB + SB]
    esems = sems[GB + SB:GB + SB + ED]
    msem = sems[GB + SB + ED]
    c = lax.axis_index("c")
    s = lax.axis_index("s")
    z0 = s * WPT

    @pl.when(s < NS - 1)
    def _():
        pltpu.async_copy(zeros_hbm.at[pl.ds(z0, WPT)],
                         acc_sh.at[pl.ds(z0, WPT)], msem)

    @pl.when(s == NS - 1)
    def _():
        pltpu.async_copy(zeros_hbm.at[pl.ds(z0, WPT_LAST)],
                         acc_sh.at[pl.ds(z0, WPT_LAST)], msem)

    chc = jnp.where(c == 0, CH0, CH1)
    row0 = c * NS * CH0 + s * chc
    pltpu.sync_copy(src_hbm.at[pl.ds(row0, GB)], src_v.at[pl.ds(0, GB)])
    pltpu.sync_copy(dst_hbm.at[pl.ds(row0, GB)], dst_v.at[pl.ds(0, GB)])
    pltpu.sync_copy(w_hbm.at[pl.ds(row0, GB)], w_v.at[pl.ds(0, GB)])
    for t in (GB, GB + 1):
        _load_erow(src_hbm, dst_hbm, w_hbm, src_v, dst_v, w_v,
                   row0, t, t % ED, esems[t % ED])

    @pl.when(s < NS - 1)
    def _():
        pltpu.make_async_copy(zeros_hbm.at[pl.ds(z0, WPT)],
                              acc_sh.at[pl.ds(z0, WPT)], msem).wait()

    @pl.when(s == NS - 1)
    def _():
        pltpu.make_async_copy(zeros_hbm.at[pl.ds(z0, WPT_LAST)],
                              acc_sh.at[pl.ds(z0, WPT_LAST)], msem).wait()

    plsc.subcore_barrier()

    for b in range(GB):
        pltpu.async_copy(g_hbm.at[src_v.at[b]], gbuf.at[b], gsems[b])

    @pl.loop(0, chc // U)
    def _(jj):
        for t in range(U):
            j = jj * U + t
            gb = t % GB
            sb = t % SB
            ql = (t + GB + 2) % ED
            qg = (t + GB) % ED
            pltpu.make_async_copy(g_hbm.at[src_v.at[t % ED]], gbuf.at[gb],
                                  gsems[gb]).wait()

            @pl.when(j >= SB)
            def _():
                pltpu.make_async_copy(sbuf.at[sb],
                                      acc_sh.at[dst_v.at[t % ED]],
                                      ssems[sb]).wait()

            @pl.when(j + GB + 2 < chc)
            def _():
                _load_erow(src_hbm, dst_hbm, w_hbm, src_v, dst_v, w_v,
                           row0, j + GB + 2, ql, esems[ql])

            @pl.loop(0, K)
            def _(i):
                wspl = plsc.load_gather(w_v.at[t % ED],
                                        [jnp.full((16,), i, jnp.int32)])
                for cb in range(D // 16):
                    sl = (i, pl.ds(cb * 16, 16))
                    sbuf.at[sb][sl] = gbuf.at[gb][sl] * wspl

            @pl.when(j + GB < chc)
            def _():
                _wait_erow(src_hbm, dst_hbm, w_hbm, src_v, dst_v, w_v,
                           row0, qg, esems[qg])
                pltpu.async_copy(g_hbm.at[src_v.at[qg]], gbuf.at[gb],
                                 gsems[gb])

            pltpu.async_copy(sbuf.at[sb], acc_sh.at[dst_v.at[t % ED]],
                             ssems[sb], add=True)

    for r in range(SB):
        pltpu.make_async_copy(sbuf.at[r], acc_sh.at[dst_v.at[r]],
                              ssems[r]).wait()
    plsc.subcore_barrier()

    @pl.when(s < NS - 1)
    def _():
        pltpu.async_copy(acc_sh.at[pl.ds(z0, WPT)],
                         out_hbm.at[c].at[pl.ds(z0, WPT)], msem).wait()

    @pl.when(s == NS - 1)
    def _():
        pltpu.async_copy(acc_sh.at[pl.ds(z0, WPT_LAST)],
                         out_hbm.at[c].at[pl.ds(z0, WPT_LAST)], msem).wait()


@jax.jit
def _sc_aggregate(g, src2d, dst2d, w2d, zeros):
    return pl.kernel(
        _agg_body,
        out_type=jax.ShapeDtypeStruct((NC, N_NODES, D), jnp.float32),
        mesh=_sc_mesh,
        scratch_types=[
            pltpu.VMEM((ED, K), jnp.int32),
            pltpu.VMEM((ED, K), jnp.int32),
            pltpu.VMEM((ED, K), jnp.float32),
            pltpu.VMEM((GB, K, D), jnp.float32),
            pltpu.VMEM((SB, K, D), jnp.float32),
            pltpu.VMEM_SHARED((N_NODES, D), jnp.float32),
        ] + [pltpu.SemaphoreType.DMA] * (GB + SB + ED + 1),
        compiler_params=_sc_params,
    )(g, src2d, dst2d, w2d, zeros)


_BLK = 2000


def _mm_body(x_ref, w_ref, o_ref):
    o_ref[...] = jnp.dot(x_ref[...], w_ref[...],
                         preferred_element_type=jnp.float32)


def _matmul(x, w):
    return pl.pallas_call(
        _mm_body,
        grid=(N_NODES // _BLK,),
        in_specs=[pl.BlockSpec((_BLK, D), lambda i: (i, 0)),
                  pl.BlockSpec((D, D), lambda i: (0, 0))],
        out_specs=pl.BlockSpec((_BLK, D), lambda i: (i, 0)),
        out_shape=jax.ShapeDtypeStruct((N_NODES, D), jnp.float32),
    )(x, w)


def _prep_body(deg_ref, h_ref, dis_ref, g_ref):
    d = deg_ref[:, 0:1] + deg_ref[:, 1:2] + 1.0
    di = lax.rsqrt(d)
    dis_ref[...] = di
    g_ref[...] = di * h_ref[...]


def _tc_prep(deg01, h):
    return pl.pallas_call(
        _prep_body,
        grid=(N_NODES // _BLK,),
        in_specs=[pl.BlockSpec((_BLK, NC), lambda i: (i, 0)),
                  pl.BlockSpec((_BLK, D), lambda i: (i, 0))],
        out_specs=[pl.BlockSpec((_BLK, 1), lambda i: (i, 0)),
                   pl.BlockSpec((_BLK, D), lambda i: (i, 0))],
        out_shape=[jax.ShapeDtypeStruct((N_NODES, 1), jnp.float32),
                   jax.ShapeDtypeStruct((N_NODES, D), jnp.float32)],
    )(deg01, h)


def _mid_body(acc_ref, g_ref, dis_ref, b_ref, w_ref, g2_ref):
    di = dis_ref[...]
    a = acc_ref[0] + acc_ref[1] + g_ref[...]
    h1 = jnp.maximum(di * a + b_ref[...], 0.0)
    g2_ref[...] = di * jnp.dot(h1, w_ref[...],
                               preferred_element_type=jnp.float32)


def _tc_mid(acc_p, g, dis, b1, W2):
    return pl.pallas_call(
        _mid_body,
        grid=(N_NODES // _BLK,),
        in_specs=[pl.BlockSpec((NC, _BLK, D), lambda i: (0, i, 0)),
                  pl.BlockSpec((_BLK, D), lambda i: (i, 0)),
                  pl.BlockSpec((_BLK, 1), lambda i: (i, 0)),
                  pl.BlockSpec((1, D), lambda i: (0, 0)),
                  pl.BlockSpec((D, D), lambda i: (0, 0))],
        out_specs=pl.BlockSpec((_BLK, D), lambda i: (i, 0)),
        out_shape=jax.ShapeDtypeStruct((N_NODES, D), jnp.float32),
    )(acc_p, g, dis, b1.reshape(1, D), W2)


def _final_body(acc_ref, g_ref, dis_ref, b_ref, o_ref):
    di = dis_ref[...]
    a = acc_ref[0] + acc_ref[1] + g_ref[...]
    o_ref[...] = di * a + b_ref[...]


def _tc_final(acc_p, g2, dis, b2):
    return pl.pallas_call(
        _final_body,
        grid=(N_NODES // _BLK,),
        in_specs=[pl.BlockSpec((NC, _BLK, D), lambda i: (0, i, 0)),
                  pl.BlockSpec((_BLK, D), lambda i: (i, 0)),
                  pl.BlockSpec((_BLK, 1), lambda i: (i, 0)),
                  pl.BlockSpec((1, D), lambda i: (0, 0))],
        out_specs=pl.BlockSpec((_BLK, D), lambda i: (i, 0)),
        out_shape=jax.ShapeDtypeStruct((N_NODES, D), jnp.float32),
    )(acc_p, g2, dis, b2.reshape(1, D))


def kernel(x, edge_index, edge_attr, W1, b1, W2, b2):
    src = edge_index[0]
    dst = edge_index[1]
    pad = E_PAD - E_EDGES
    src2d = jnp.concatenate(
        [src, jnp.zeros((pad,), jnp.int32)]).reshape(NW * CH, K)
    dst2d = jnp.concatenate(
        [dst, jnp.zeros((pad,), jnp.int32)]).reshape(NW * CH, K)
    w2d = jnp.concatenate(
        [edge_attr, jnp.zeros((pad,), jnp.float32)]).reshape(NW * CH, K)
    zeros = jnp.zeros((N_NODES, D), jnp.float32)
    zd = jnp.zeros((N_DEG,), jnp.float32)

    deg01 = _sc_degree(dst2d, w2d, zd)[:, :N_NODES].T
    h = _matmul(x, W1)
    dis, g1 = _tc_prep(deg01, h)

    acc1 = _sc_aggregate(g1, src2d, dst2d, w2d, zeros)
    g2 = _tc_mid(acc1, g1, dis, b1, W2)
    acc2 = _sc_aggregate(g2, src2d, dst2d, w2d, zeros)
    return _tc_final(acc2, g2, dis, b2)


# speedup vs baseline: 1.1431x; 1.0415x over previous
import dataclasses
import functools

import jax
import jax.numpy as jnp
from jax import lax
from jax.experimental import pallas as pl
from jax.experimental.pallas import tpu as pltpu
from jax.experimental.pallas import tpu_sc as plsc

N_NODES = 10000
D = 128
E_EDGES = 320000

NC, NS = 2, 16
NW = NC * NS
K = 56
CH0 = 200
CH1 = 160
CH = (CH0 + CH1) // 2
GB = 4
SB = 2
ED = 8
U = 8
E_PAD = NW * CH * K
WPT = 632
WPT_LAST = N_NODES - 15 * WPT
N_DEG = 10240
DEG_PER_TILE = N_DEG // NS

_sc_mesh = plsc.VectorSubcoreMesh(core_axis_name="c", subcore_axis_name="s")

_sc_params = pltpu.CompilerParams()
if "needs_layout_passes" in pltpu.CompilerParams.__dataclass_fields__:
    _sc_params = dataclasses.replace(_sc_params, needs_layout_passes=False)


def _deg_body(dst_hbm, w_hbm, zd_hbm, out_hbm, dst_v, w_v, deg_sh, sem):
    c = lax.axis_index("c")
    s = lax.axis_index("s")
    z0 = s * DEG_PER_TILE
    pltpu.async_copy(zd_hbm.at[pl.ds(z0, DEG_PER_TILE)],
                     deg_sh.at[pl.ds(z0, DEG_PER_TILE)], sem).wait()
    chc = jnp.where(c == 0, CH0, CH1)
    row0 = c * NS * CH0 + s * chc

    @pl.when(c == 0)
    def _():
        pltpu.sync_copy(dst_hbm.at[pl.ds(row0, CH0)], dst_v.at[pl.ds(0, CH0)])
        pltpu.sync_copy(w_hbm.at[pl.ds(row0, CH0)], w_v.at[pl.ds(0, CH0)])

    @pl.when(c == 1)
    def _():
        pltpu.sync_copy(dst_hbm.at[pl.ds(row0, CH1)], dst_v.at[pl.ds(0, CH1)])
        pltpu.sync_copy(w_hbm.at[pl.ds(row0, CH1)], w_v.at[pl.ds(0, CH1)])

    plsc.subcore_barrier()

    @pl.loop(0, chc)
    def _(j):
        pltpu.sync_copy(w_v.at[j], deg_sh.at[dst_v.at[j]], add=True)

    plsc.subcore_barrier()
    pltpu.async_copy(deg_sh.at[pl.ds(z0, DEG_PER_TILE)],
                     out_hbm.at[c].at[pl.ds(z0, DEG_PER_TILE)], sem).wait()


@jax.jit
def _sc_degree(dst2d, w2d, zd):
    return pl.kernel(
        _deg_body,
        out_type=jax.ShapeDtypeStruct((NC, N_DEG), jnp.float32),
        mesh=_sc_mesh,
        scratch_types=[
            pltpu.VMEM((CH0, K), jnp.int32),
            pltpu.VMEM((CH0, K), jnp.float32),
            pltpu.VMEM_SHARED((N_DEG,), jnp.float32),
            pltpu.SemaphoreType.DMA,
        ],
    )(dst2d, w2d, zd)


def _load_erow(src_hbm, dst_hbm, w_hbm, src_v, dst_v, w_v, row0, j, q, sem):
    pltpu.async_copy(src_hbm.at[pl.ds(row0 + j, 1)],
                     src_v.at[pl.ds(q, 1)], sem)
    pltpu.async_copy(dst_hbm.at[pl.ds(row0 + j, 1)],
                     dst_v.at[pl.ds(q, 1)], sem)
    pltpu.async_copy(w_hbm.at[pl.ds(row0 + j, 1)],
                     w_v.at[pl.ds(q, 1)], sem)


def _wait_erow(src_hbm, dst_hbm, w_hbm, src_v, dst_v, w_v, row0, q, sem):
    pltpu.make_async_copy(src_hbm.at[pl.ds(row0, 1)],
                          src_v.at[pl.ds(q, 1)], sem).wait()
    pltpu.make_async_copy(dst_hbm.at[pl.ds(row0, 1)],
                          dst_v.at[pl.ds(q, 1)], sem).wait()
    pltpu.make_async_copy(w_hbm.at[pl.ds(row0, 1)],
                          w_v.at[pl.ds(q, 1)], sem).wait()


def _agg_body(g_hbm, src_hbm, dst_hbm, w_hbm, zeros_hbm, out_hbm,
              src_v, dst_v, w_v, gbuf, sbuf, acc_sh, *sems):
    gsems = sems[0:GB]
    ssems = sems[GB:GB + SB]
    esems = sems[GB + SB:GB + SB + ED]
    msem = sems[GB + SB + ED]
    c = lax.axis_index("c")
    s = lax.axis_index("s")
    z0 = s * WPT

    @pl.when(s < NS - 1)
    def _():
        pltpu.async_copy(zeros_hbm.at[pl.ds(z0, WPT)],
                         acc_sh.at[pl.ds(z0, WPT)], msem)

    @pl.when(s == NS - 1)
    def _():
        pltpu.async_copy(zeros_hbm.at[pl.ds(z0, WPT_LAST)],
                         acc_sh.at[pl.ds(z0, WPT_LAST)], msem)

    chc = jnp.where(c == 0, CH0, CH1)
    row0 = c * NS * CH0 + s * chc
    pltpu.sync_copy(src_hbm.at[pl.ds(row0, GB)], src_v.at[pl.ds(0, GB)])
    pltpu.sync_copy(dst_hbm.at[pl.ds(row0, GB)], dst_v.at[pl.ds(0, GB)])
    pltpu.sync_copy(w_hbm.at[pl.ds(row0, GB)], w_v.at[pl.ds(0, GB)])
    for t in (GB, GB + 1):
        _load_erow(src_hbm, dst_hbm, w_hbm, src_v, dst_v, w_v,
                   row0, t, t % ED, esems[t % ED])

    @pl.when(s < NS - 1)
    def _():
        pltpu.make_async_copy(zeros_hbm.at[pl.ds(z0, WPT)],
                              acc_sh.at[pl.ds(z0, WPT)], msem).wait()

    @pl.when(s == NS - 1)
    def _():
        pltpu.make_async_copy(zeros_hbm.at[pl.ds(z0, WPT_LAST)],
                              acc_sh.at[pl.ds(z0, WPT_LAST)], msem).wait()

    plsc.subcore_barrier()

    for b in range(GB):
        pltpu.async_copy(g_hbm.at[src_v.at[b]], gbuf.at[b], gsems[b])

    @pl.loop(0, chc // U)
    def _(jj):
        for t in range(U):
            j = jj * U + t
            gb = t % GB
            sb = t % SB
            ql = (t + GB + 2) % ED
            qg = (t + GB) % ED
            pltpu.make_async_copy(g_hbm.at[src_v.at[t % ED]], gbuf.at[gb],
                                  gsems[gb]).wait()

            @pl.when(j >= SB)
            def _():
                pltpu.make_async_copy(sbuf.at[sb],
                                      acc_sh.at[dst_v.at[t % ED]],
                                      ssems[sb]).wait()

            @pl.when(j + GB + 2 < chc)
            def _():
                _load_erow(src_hbm, dst_hbm, w_hbm, src_v, dst_v, w_v,
                           row0, j + GB + 2, ql, esems[ql])

            @pl.loop(0, K)
            def _(i):
                wspl = plsc.load_gather(w_v.at[t % ED],
                                        [jnp.full((16,), i, jnp.int32)])
                for cb in range(D // 16):
                    sl = (i, pl.ds(cb * 16, 16))
                    sbuf.at[sb][sl] = gbuf.at[gb][sl] * wspl

            @pl.when(j + GB < chc)
            def _():
                _wait_erow(src_hbm, dst_hbm, w_hbm, src_v, dst_v, w_v,
                           row0, qg, esems[qg])
                pltpu.async_copy(g_hbm.at[src_v.at[qg]], gbuf.at[gb],
                                 gsems[gb])

            pltpu.async_copy(sbuf.at[sb], acc_sh.at[dst_v.at[t % ED]],
                             ssems[sb], add=True)

    for r in range(SB):
        pltpu.make_async_copy(sbuf.at[r], acc_sh.at[dst_v.at[r]],
                              ssems[r]).wait()
    plsc.subcore_barrier()

    @pl.when(s < NS - 1)
    def _():
        pltpu.async_copy(acc_sh.at[pl.ds(z0, WPT)],
                         out_hbm.at[c].at[pl.ds(z0, WPT)], msem).wait()

    @pl.when(s == NS - 1)
    def _():
        pltpu.async_copy(acc_sh.at[pl.ds(z0, WPT_LAST)],
                         out_hbm.at[c].at[pl.ds(z0, WPT_LAST)], msem).wait()


@jax.jit
def _sc_aggregate(g, src2d, dst2d, w2d, zeros):
    return pl.kernel(
        _agg_body,
        out_type=jax.ShapeDtypeStruct((NC, N_NODES, D), jnp.float32),
        mesh=_sc_mesh,
        scratch_types=[
            pltpu.VMEM((ED, K), jnp.int32),
            pltpu.VMEM((ED, K), jnp.int32),
            pltpu.VMEM((ED, K), jnp.float32),
            pltpu.VMEM((GB, K, D), jnp.float32),
            pltpu.VMEM((SB, K, D), jnp.float32),
            pltpu.VMEM_SHARED((N_NODES, D), jnp.float32),
        ] + [pltpu.SemaphoreType.DMA] * (GB + SB + ED + 1),
        compiler_params=_sc_params,
    )(g, src2d, dst2d, w2d, zeros)


_BLK = 2000


def _mm_body(x_ref, w_ref, o_ref):
    o_ref[...] = jnp.dot(x_ref[...], w_ref[...],
                         preferred_element_type=jnp.float32)


def _matmul(x, w):
    return pl.pallas_call(
        _mm_body,
        grid=(N_NODES // _BLK,),
        in_specs=[pl.BlockSpec((_BLK, D), lambda i: (i, 0)),
                  pl.BlockSpec((D, D), lambda i: (0, 0))],
        out_specs=pl.BlockSpec((_BLK, D), lambda i: (i, 0)),
        out_shape=jax.ShapeDtypeStruct((N_NODES, D), jnp.float32),
    )(x, w)


def _prep_body(deg_ref, h_ref, dis_ref, g_ref):
    d = deg_ref[:, 0:1] + deg_ref[:, 1:2] + 1.0
    di = lax.rsqrt(d)
    dis_ref[...] = di
    g_ref[...] = di * h_ref[...]


def _tc_prep(deg01, h):
    return pl.pallas_call(
        _prep_body,
        grid=(N_NODES // _BLK,),
        in_specs=[pl.BlockSpec((_BLK, NC), lambda i: (i, 0)),
                  pl.BlockSpec((_BLK, D), lambda i: (i, 0))],
        out_specs=[pl.BlockSpec((_BLK, 1), lambda i: (i, 0)),
                   pl.BlockSpec((_BLK, D), lambda i: (i, 0))],
        out_shape=[jax.ShapeDtypeStruct((N_NODES, 1), jnp.float32),
                   jax.ShapeDtypeStruct((N_NODES, D), jnp.float32)],
    )(deg01, h)


def _mid_body(acc_ref, g_ref, dis_ref, b_ref, w_ref, g2_ref):
    di = dis_ref[...]
    a = acc_ref[0] + acc_ref[1] + g_ref[...]
    h1 = jnp.maximum(di * a + b_ref[...], 0.0)
    g2_ref[...] = di * jnp.dot(h1, w_ref[...],
                               preferred_element_type=jnp.float32)


def _tc_mid(acc_p, g, dis, b1, W2):
    return pl.pallas_call(
        _mid_body,
        grid=(N_NODES // _BLK,),
        in_specs=[pl.BlockSpec((NC, _BLK, D), lambda i: (0, i, 0)),
                  pl.BlockSpec((_BLK, D), lambda i: (i, 0)),
                  pl.BlockSpec((_BLK, 1), lambda i: (i, 0)),
                  pl.BlockSpec((1, D), lambda i: (0, 0)),
                  pl.BlockSpec((D, D), lambda i: (0, 0))],
        out_specs=pl.BlockSpec((_BLK, D), lambda i: (i, 0)),
        out_shape=jax.ShapeDtypeStruct((N_NODES, D), jnp.float32),
    )(acc_p, g, dis, b1.reshape(1, D), W2)


def _final_body(acc_ref, g_ref, dis_ref, b_ref, o_ref):
    di = dis_ref[...]
    a = acc_ref[0] + acc_ref[1] + g_ref[...]
    o_ref[...] = di * a + b_ref[...]


def _tc_final(acc_p, g2, dis, b2):
    return pl.pallas_call(
        _final_body,
        grid=(N_NODES // _BLK,),
        in_specs=[pl.BlockSpec((NC, _BLK, D), lambda i: (0, i, 0)),
                  pl.BlockSpec((_BLK, D), lambda i: (i, 0)),
                  pl.BlockSpec((_BLK, 1), lambda i: (i, 0)),
                  pl.BlockSpec((1, D), lambda i: (0, 0))],
        out_specs=pl.BlockSpec((_BLK, D), lambda i: (i, 0)),
        out_shape=jax.ShapeDtypeStruct((N_NODES, D), jnp.float32),
    )(acc_p, g2, dis, b2.reshape(1, D))


def kernel(x, edge_index, edge_attr, W1, b1, W2, b2):
    src = edge_index[0]
    dst = edge_index[1]
    pad = E_PAD - E_EDGES
    src2d = jnp.concatenate(
        [src, jnp.zeros((pad,), jnp.int32)]).reshape(NW * CH, K)
    dst2d = jnp.concatenate(
        [dst, jnp.zeros((pad,), jnp.int32)]).reshape(NW * CH, K)
    w2d = jnp.concatenate(
        [edge_attr, jnp.zeros((pad,), jnp.float32)]).reshape(NW * CH, K)
    zeros = jnp.zeros((N_NODES, D), jnp.float32)
    zd = jnp.zeros((N_DEG,), jnp.float32)

    deg01 = _sc_degree(dst2d, w2d, zd)[:, :N_NODES].T
    h = _matmul(x, W1)
    dis, g1 = _tc_prep(deg01, h)

    acc1 = _sc_aggregate(g1, src2d, dst2d, w2d, zeros)
    g2 = _tc_mid(acc1, g1, dis, b1, W2)
    acc2 = _sc_aggregate(g2, src2d, dst2d, w2d, zeros)
    return _tc_final(acc2, g2, dis, b2)
